# baseline (speedup 1.0000x reference)
import functools

import jax
import jax.numpy as jnp
from jax import lax
from jax.experimental import pallas as pl
from jax.experimental.pallas import tpu as pltpu
from jax.experimental.pallas import tpu_sc as plsc

N_NODES = 10000
N_EDGES = 160000
IN_DIM = 512
OUT_DIM = 256
HALF = 128
NC = 2
NS = 16
CHUNK = 128
CHUNKS_PER_TILE = 80
EDGES_PER_TILE = CHUNK * CHUNKS_PER_TILE
E_PAD = EDGES_PER_TILE * NS
NODES_PER_TILE = 624
NODES_LAST_TILE = N_NODES - NODES_PER_TILE * (NS - 1)
ACC_ROWS = N_NODES + 16
ROW_BLK = 2000


def _mm_body(x_ref, w0_ref, w1_ref, o0_ref, o1_ref):
    o0_ref[...] = jnp.dot(x_ref[...], w0_ref[...],
                          preferred_element_type=jnp.float32)
    o1_ref[...] = jnp.dot(x_ref[...], w1_ref[...],
                          preferred_element_type=jnp.float32)


def _linear(x, w0, w1):
    return pl.pallas_call(
        _mm_body,
        grid=(N_NODES // ROW_BLK,),
        in_specs=[
            pl.BlockSpec((ROW_BLK, IN_DIM), lambda r: (r, 0)),
            pl.BlockSpec((IN_DIM, HALF), lambda r: (0, 0)),
            pl.BlockSpec((IN_DIM, HALF), lambda r: (0, 0)),
        ],
        out_specs=[
            pl.BlockSpec((ROW_BLK, HALF), lambda r: (r, 0)),
            pl.BlockSpec((ROW_BLK, HALF), lambda r: (r, 0)),
        ],
        out_shape=[
            jax.ShapeDtypeStruct((N_NODES, HALF), jnp.float32),
            jax.ShapeDtypeStruct((N_NODES, HALF), jnp.float32),
        ],
    )(x, w0, w1)


_mesh = plsc.VectorSubcoreMesh(core_axis_name="c", subcore_axis_name="s",
                               num_cores=NC, num_subcores=NS)


@functools.partial(
    pl.kernel,
    out_type=jax.ShapeDtypeStruct((N_NODES, OUT_DIM), jnp.float32),
    mesh=_mesh,
    scratch_types=[
        pltpu.VMEM((CHUNKS_PER_TILE, CHUNK), jnp.int32),
        pltpu.VMEM((CHUNKS_PER_TILE, CHUNK), jnp.int32),
        pltpu.VMEM((CHUNK, HALF), jnp.float32),
        pltpu.VMEM_SHARED((ACC_ROWS, HALF), jnp.float32),
        pltpu.SemaphoreType.DMA,
    ],
)
def _aggregate(h0, h1, src3, dst3, zeros, out,
               src_v, dst_v, rows_v, acc, sem):
    cid = lax.axis_index("c")
    sid = lax.axis_index("s")
    node0 = sid * NODES_PER_TILE

    @pl.when(sid < NS - 1)
    def _():
        pltpu.sync_copy(zeros.at[pl.ds(node0, NODES_PER_TILE)],
                        acc.at[pl.ds(node0, NODES_PER_TILE)])

    @pl.when(sid == NS - 1)
    def _():
        pltpu.sync_copy(zeros.at[pl.ds(node0, NODES_LAST_TILE)],
                        acc.at[pl.ds(node0, NODES_LAST_TILE)])

    pltpu.sync_copy(src3.at[sid], src_v)
    pltpu.sync_copy(dst3.at[sid], dst_v)
    plsc.subcore_barrier()

    def chunk(j, carry):
        @pl.when(cid == 0)
        def _():
            pltpu.async_copy(h0.at[src_v.at[j]], rows_v, sem).wait()

        @pl.when(cid == 1)
        def _():
            pltpu.async_copy(h1.at[src_v.at[j]], rows_v, sem).wait()

        pltpu.sync_copy(rows_v, acc.at[dst_v.at[j]], add=True)
        return carry

    lax.fori_loop(0, CHUNKS_PER_TILE, chunk, 0)
    plsc.subcore_barrier()

    col0 = pl.multiple_of(cid * HALF, HALF)

    @pl.when(sid < NS - 1)
    def _():
        pltpu.sync_copy(acc.at[pl.ds(node0, NODES_PER_TILE)],
                        out.at[pl.ds(node0, NODES_PER_TILE),
                               pl.ds(col0, HALF)])

    @pl.when(sid == NS - 1)
    def _():
        pltpu.sync_copy(acc.at[pl.ds(node0, NODES_LAST_TILE)],
                        out.at[pl.ds(node0, NODES_LAST_TILE),
                               pl.ds(col0, HALF)])


def kernel(x, edge_index, W):
    ei = edge_index.astype(jnp.int32)
    src = ei[1]
    dst = ei[0]
    pad = E_PAD - N_EDGES
    src3 = jnp.concatenate(
        [src, jnp.zeros((pad,), jnp.int32)]).reshape(NS, CHUNKS_PER_TILE, CHUNK)
    dst3 = jnp.concatenate(
        [dst, jnp.full((pad,), N_NODES, jnp.int32)]).reshape(NS, CHUNKS_PER_TILE, CHUNK)
    wt = W.T
    h0, h1 = _linear(x, wt[:, :HALF], wt[:, HALF:])
    zeros = jnp.zeros((N_NODES, HALF), jnp.float32)
    return _aggregate(h0, h1, src3, dst3, zeros)


# speedup vs baseline: 1.0795x; 1.0795x over previous
import functools

import jax
import jax.numpy as jnp
from jax import lax
from jax.experimental import pallas as pl
from jax.experimental.pallas import tpu as pltpu
from jax.experimental.pallas import tpu_sc as plsc

N_NODES = 10000
N_EDGES = 160000
IN_DIM = 512
OUT_DIM = 256
HALF = 128
NC = 2
NS = 16
CHUNK = 128
CHUNKS_PER_TILE = 80
EDGES_PER_TILE = CHUNK * CHUNKS_PER_TILE
E_PAD = EDGES_PER_TILE * NS
NODES_PER_TILE = 624
NODES_LAST_TILE = N_NODES - NODES_PER_TILE * (NS - 1)
ACC_ROWS = N_NODES + 16
ROW_BLK = 2000


def _mm_body(x_ref, w0_ref, w1_ref, o0_ref, o1_ref):
    o0_ref[...] = jnp.dot(x_ref[...], w0_ref[...],
                          preferred_element_type=jnp.float32)
    o1_ref[...] = jnp.dot(x_ref[...], w1_ref[...],
                          preferred_element_type=jnp.float32)


def _linear(x, w0, w1):
    return pl.pallas_call(
        _mm_body,
        grid=(N_NODES // ROW_BLK,),
        in_specs=[
            pl.BlockSpec((ROW_BLK, IN_DIM), lambda r: (r, 0)),
            pl.BlockSpec((IN_DIM, HALF), lambda r: (0, 0)),
            pl.BlockSpec((IN_DIM, HALF), lambda r: (0, 0)),
        ],
        out_specs=[
            pl.BlockSpec((ROW_BLK, HALF), lambda r: (r, 0)),
            pl.BlockSpec((ROW_BLK, HALF), lambda r: (r, 0)),
        ],
        out_shape=[
            jax.ShapeDtypeStruct((N_NODES, HALF), jnp.float32),
            jax.ShapeDtypeStruct((N_NODES, HALF), jnp.float32),
        ],
    )(x, w0, w1)


_mesh = plsc.VectorSubcoreMesh(core_axis_name="c", subcore_axis_name="s",
                               num_cores=NC, num_subcores=NS)


@functools.partial(
    pl.kernel,
    out_type=(jax.ShapeDtypeStruct((N_NODES, HALF), jnp.float32),
              jax.ShapeDtypeStruct((N_NODES, HALF), jnp.float32)),
    mesh=_mesh,
    scratch_types=[
        pltpu.VMEM((CHUNKS_PER_TILE, CHUNK), jnp.int32),
        pltpu.VMEM((CHUNKS_PER_TILE, CHUNK), jnp.int32),
        pltpu.VMEM((CHUNK, HALF), jnp.float32),
        pltpu.VMEM_SHARED((ACC_ROWS, HALF), jnp.float32),
        pltpu.SemaphoreType.DMA,
    ],
)
def _aggregate(h0, h1, src3, dst3, zeros, o0, o1,
               src_v, dst_v, rows_v, acc, sem):
    cid = lax.axis_index("c")
    sid = lax.axis_index("s")
    node0 = sid * NODES_PER_TILE

    @pl.when(sid < NS - 1)
    def _():
        pltpu.sync_copy(zeros.at[pl.ds(node0, NODES_PER_TILE)],
                        acc.at[pl.ds(node0, NODES_PER_TILE)])

    @pl.when(sid == NS - 1)
    def _():
        pltpu.sync_copy(zeros.at[pl.ds(node0, NODES_LAST_TILE)],
                        acc.at[pl.ds(node0, NODES_LAST_TILE)])

    pltpu.sync_copy(src3.at[sid], src_v)
    pltpu.sync_copy(dst3.at[sid], dst_v)
    plsc.subcore_barrier()

    def chunk(j, carry):
        @pl.when(cid == 0)
        def _():
            pltpu.async_copy(h0.at[src_v.at[j]], rows_v, sem).wait()

        @pl.when(cid == 1)
        def _():
            pltpu.async_copy(h1.at[src_v.at[j]], rows_v, sem).wait()

        pltpu.sync_copy(rows_v, acc.at[dst_v.at[j]], add=True)
        return carry

    lax.fori_loop(0, CHUNKS_PER_TILE, chunk, 0)
    plsc.subcore_barrier()

    @pl.when((cid == 0) & (sid < NS - 1))
    def _():
        pltpu.sync_copy(acc.at[pl.ds(node0, NODES_PER_TILE)],
                        o0.at[pl.ds(node0, NODES_PER_TILE)])

    @pl.when((cid == 0) & (sid == NS - 1))
    def _():
        pltpu.sync_copy(acc.at[pl.ds(node0, NODES_LAST_TILE)],
                        o0.at[pl.ds(node0, NODES_LAST_TILE)])

    @pl.when((cid == 1) & (sid < NS - 1))
    def _():
        pltpu.sync_copy(acc.at[pl.ds(node0, NODES_PER_TILE)],
                        o1.at[pl.ds(node0, NODES_PER_TILE)])

    @pl.when((cid == 1) & (sid == NS - 1))
    def _():
        pltpu.sync_copy(acc.at[pl.ds(node0, NODES_LAST_TILE)],
                        o1.at[pl.ds(node0, NODES_LAST_TILE)])


def kernel(x, edge_index, W):
    ei = edge_index.astype(jnp.int32)
    src = ei[1]
    dst = ei[0]
    pad = E_PAD - N_EDGES
    src3 = jnp.concatenate(
        [src, jnp.zeros((pad,), jnp.int32)]).reshape(NS, CHUNKS_PER_TILE, CHUNK)
    dst3 = jnp.concatenate(
        [dst, jnp.full((pad,), N_NODES, jnp.int32)]).reshape(NS, CHUNKS_PER_TILE, CHUNK)
    wt = W.T
    h0, h1 = _linear(x, wt[:, :HALF], wt[:, HALF:])
    zeros = jnp.zeros((N_NODES, HALF), jnp.float32)
    o0, o1 = _aggregate(h0, h1, src3, dst3, zeros)
    return jnp.concatenate([o0, o1], axis=1)


# speedup vs baseline: 1.8992x; 1.7593x over previous
import functools

import jax
import jax.numpy as jnp
from jax import lax
from jax.experimental import pallas as pl
from jax.experimental.pallas import tpu as pltpu
from jax.experimental.pallas import tpu_sc as plsc

N_NODES = 10000
N_EDGES = 160000
IN_DIM = 512
OUT_DIM = 256
HALF = 128
NC = 2
NS = 16
CHUNK = 128
CHUNKS_PER_TILE = 80
EDGES_PER_TILE = CHUNK * CHUNKS_PER_TILE
E_PAD = EDGES_PER_TILE * NS
NODES_PER_TILE = 624
NODES_LAST_TILE = N_NODES - NODES_PER_TILE * (NS - 1)
ACC_ROWS = N_NODES + 16
ROW_BLK = 2000


def _mm_body(x_ref, w0_ref, w1_ref, o0_ref, o1_ref):
    o0_ref[...] = jnp.dot(x_ref[...], w0_ref[...],
                          preferred_element_type=jnp.float32)
    o1_ref[...] = jnp.dot(x_ref[...], w1_ref[...],
                          preferred_element_type=jnp.float32)


def _linear(x, w0, w1):
    return pl.pallas_call(
        _mm_body,
        grid=(N_NODES // ROW_BLK,),
        in_specs=[
            pl.BlockSpec((ROW_BLK, IN_DIM), lambda r: (r, 0)),
            pl.BlockSpec((IN_DIM, HALF), lambda r: (0, 0)),
            pl.BlockSpec((IN_DIM, HALF), lambda r: (0, 0)),
        ],
        out_specs=[
            pl.BlockSpec((ROW_BLK, HALF), lambda r: (r, 0)),
            pl.BlockSpec((ROW_BLK, HALF), lambda r: (r, 0)),
        ],
        out_shape=[
            jax.ShapeDtypeStruct((N_NODES, HALF), jnp.float32),
            jax.ShapeDtypeStruct((N_NODES, HALF), jnp.float32),
        ],
    )(x, w0, w1)


_mesh = plsc.VectorSubcoreMesh(core_axis_name="c", subcore_axis_name="s",
                               num_cores=NC, num_subcores=NS)


@functools.partial(
    pl.kernel,
    out_type=(jax.ShapeDtypeStruct((N_NODES, HALF), jnp.float32),
              jax.ShapeDtypeStruct((N_NODES, HALF), jnp.float32)),
    mesh=_mesh,
    scratch_types=[
        pltpu.VMEM((CHUNKS_PER_TILE, CHUNK), jnp.int32),
        pltpu.VMEM((CHUNKS_PER_TILE, CHUNK), jnp.int32),
        pltpu.VMEM((CHUNK, HALF), jnp.float32),
        pltpu.VMEM_SHARED((ACC_ROWS, HALF), jnp.float32),
        pltpu.SemaphoreType.DMA,
    ],
)
def _aggregate(h0, h1, src3, dst3, zeros, o0, o1,
               src_v, dst_v, rows_v, acc, sem):
    cid = lax.axis_index("c")
    sid = lax.axis_index("s")
    node0 = sid * NODES_PER_TILE

    @pl.when(sid < NS - 1)
    def _():
        pltpu.sync_copy(zeros.at[pl.ds(node0, NODES_PER_TILE)],
                        acc.at[pl.ds(node0, NODES_PER_TILE)])

    @pl.when(sid == NS - 1)
    def _():
        pltpu.sync_copy(zeros.at[pl.ds(node0, NODES_LAST_TILE)],
                        acc.at[pl.ds(node0, NODES_LAST_TILE)])

    pltpu.sync_copy(src3.at[sid], src_v)
    pltpu.sync_copy(dst3.at[sid], dst_v)
    plsc.subcore_barrier()

    def chunk(j, carry):
        @pl.when(cid == 0)
        def _():
            pltpu.async_copy(h0.at[src_v.at[j]], rows_v, sem).wait()

        @pl.when(cid == 1)
        def _():
            pltpu.async_copy(h1.at[src_v.at[j]], rows_v, sem).wait()

        pltpu.sync_copy(rows_v, acc.at[dst_v.at[j]], add=True)
        return carry

    lax.fori_loop(0, CHUNKS_PER_TILE, chunk, 0)
    plsc.subcore_barrier()

    @pl.when((cid == 0) & (sid < NS - 1))
    def _():
        pltpu.sync_copy(acc.at[pl.ds(node0, NODES_PER_TILE)],
                        o0.at[pl.ds(node0, NODES_PER_TILE)])

    @pl.when((cid == 0) & (sid == NS - 1))
    def _():
        pltpu.sync_copy(acc.at[pl.ds(node0, NODES_LAST_TILE)],
                        o0.at[pl.ds(node0, NODES_LAST_TILE)])

    @pl.when((cid == 1) & (sid < NS - 1))
    def _():
        pltpu.sync_copy(acc.at[pl.ds(node0, NODES_PER_TILE)],
                        o1.at[pl.ds(node0, NODES_PER_TILE)])

    @pl.when((cid == 1) & (sid == NS - 1))
    def _():
        pltpu.sync_copy(acc.at[pl.ds(node0, NODES_LAST_TILE)],
                        o1.at[pl.ds(node0, NODES_LAST_TILE)])


def kernel(x, edge_index, W):
    ei = edge_index.astype(jnp.int32)
    src = ei[1]
    dst = ei[0]
    ept = N_EDGES // NS
    padt = EDGES_PER_TILE - ept
    pad_src = jnp.broadcast_to(jnp.arange(padt, dtype=jnp.int32), (NS, padt))
    pad_dst = jnp.broadcast_to(
        N_NODES + (jnp.arange(padt, dtype=jnp.int32) % 16), (NS, padt))
    src3 = jnp.concatenate(
        [src.reshape(NS, ept), pad_src], axis=1).reshape(
            NS, CHUNKS_PER_TILE, CHUNK)
    dst3 = jnp.concatenate(
        [dst.reshape(NS, ept), pad_dst], axis=1).reshape(
            NS, CHUNKS_PER_TILE, CHUNK)
    wt = W.T
    h0, h1 = _linear(x, wt[:, :HALF], wt[:, HALF:])
    zeros = jnp.zeros((N_NODES, HALF), jnp.float32)
    o0, o1 = _aggregate(h0, h1, src3, dst3, zeros)
    return jnp.concatenate([o0, o1], axis=1)


# speedup vs baseline: 2.6705x; 1.4061x over previous
import functools

import jax
import jax.numpy as jnp
from jax import lax
from jax.experimental import pallas as pl
from jax.experimental.pallas import tpu as pltpu
from jax.experimental.pallas import tpu_sc as plsc

N_NODES = 10000
N_EDGES = 160000
IN_DIM = 512
OUT_DIM = 256
HALF = 128
NC = 2
NS = 16
CHUNK = 128
CHUNKS_PER_TILE = 80
PASSES = 2
CPP = CHUNKS_PER_TILE // PASSES
EDGES_PER_TILE = CHUNK * CHUNKS_PER_TILE
E_PAD = EDGES_PER_TILE * NS
NODES_PER_TILE = 624
NODES_LAST_TILE = N_NODES - NODES_PER_TILE * (NS - 1)
ACC_ROWS = N_NODES + 16
ROW_BLK = 2000


def _mm_body(x_ref, w0_ref, w1_ref, o0_ref, o1_ref):
    o0_ref[...] = jnp.dot(x_ref[...], w0_ref[...],
                          preferred_element_type=jnp.float32)
    o1_ref[...] = jnp.dot(x_ref[...], w1_ref[...],
                          preferred_element_type=jnp.float32)


def _linear(x, w0, w1):
    return pl.pallas_call(
        _mm_body,
        grid=(N_NODES // ROW_BLK,),
        in_specs=[
            pl.BlockSpec((ROW_BLK, IN_DIM), lambda r: (r, 0)),
            pl.BlockSpec((IN_DIM, HALF), lambda r: (0, 0)),
            pl.BlockSpec((IN_DIM, HALF), lambda r: (0, 0)),
        ],
        out_specs=[
            pl.BlockSpec((ROW_BLK, HALF), lambda r: (r, 0)),
            pl.BlockSpec((ROW_BLK, HALF), lambda r: (r, 0)),
        ],
        out_shape=[
            jax.ShapeDtypeStruct((N_NODES, HALF), jnp.float32),
            jax.ShapeDtypeStruct((N_NODES, HALF), jnp.float32),
        ],
    )(x, w0, w1)


_mesh = plsc.VectorSubcoreMesh(core_axis_name="c", subcore_axis_name="s",
                               num_cores=NC, num_subcores=NS)


@functools.partial(
    pl.kernel,
    out_type=(jax.ShapeDtypeStruct((N_NODES, HALF), jnp.float32),
              jax.ShapeDtypeStruct((N_NODES, HALF), jnp.float32)),
    mesh=_mesh,
    scratch_types=[
        pltpu.VMEM((CPP, CHUNK), jnp.int32),
        pltpu.VMEM((CPP, CHUNK), jnp.int32),
        pltpu.VMEM((CHUNK, HALF), jnp.float32),
        pltpu.VMEM((CHUNK, HALF), jnp.float32),
        pltpu.VMEM_SHARED((ACC_ROWS, HALF), jnp.float32),
        pltpu.SemaphoreType.DMA,
        pltpu.SemaphoreType.DMA,
    ],
)
def _aggregate(h0, h1, src3, dst3, zeros, o0, o1,
               src_v, dst_v, r0, r1, acc, g0, g1):
    bufs = (r0, r1)
    gsems = (g0, g1)
    cid = lax.axis_index("c")
    sid = lax.axis_index("s")
    node0 = sid * NODES_PER_TILE

    @pl.when(sid < NS - 1)
    def _():
        pltpu.sync_copy(zeros.at[pl.ds(node0, NODES_PER_TILE)],
                        acc.at[pl.ds(node0, NODES_PER_TILE)])

    @pl.when(sid == NS - 1)
    def _():
        pltpu.sync_copy(zeros.at[pl.ds(node0, NODES_LAST_TILE)],
                        acc.at[pl.ds(node0, NODES_LAST_TILE)])

    plsc.subcore_barrier()

    def start_gather(jj, b):
        @pl.when(cid == 0)
        def _():
            pltpu.async_copy(h0.at[src_v.at[jj]], bufs[b], gsems[b])

        @pl.when(cid == 1)
        def _():
            pltpu.async_copy(h1.at[src_v.at[jj]], bufs[b], gsems[b])

    def wait_gather(b):
        pltpu.make_async_copy(h0.at[pl.ds(0, CHUNK)], bufs[b], gsems[b]).wait()

    for p in range(PASSES):
        pltpu.sync_copy(src3.at[sid, pl.ds(p * CPP, CPP)], src_v)
        pltpu.sync_copy(dst3.at[sid, pl.ds(p * CPP, CPP)], dst_v)
        start_gather(0, 0)

        def pair(i, carry):
            for b in range(2):
                j = 2 * i + b

                @pl.when(j + 1 < CPP)
                def _(j=j, b=b):
                    start_gather(j + 1, 1 - b)

                wait_gather(b)
                pltpu.sync_copy(bufs[b], acc.at[dst_v.at[j]], add=True)
            return carry

        lax.fori_loop(0, CPP // 2, pair, 0)

    plsc.subcore_barrier()

    @pl.when((cid == 0) & (sid < NS - 1))
    def _():
        pltpu.sync_copy(acc.at[pl.ds(node0, NODES_PER_TILE)],
                        o0.at[pl.ds(node0, NODES_PER_TILE)])

    @pl.when((cid == 0) & (sid == NS - 1))
    def _():
        pltpu.sync_copy(acc.at[pl.ds(node0, NODES_LAST_TILE)],
                        o0.at[pl.ds(node0, NODES_LAST_TILE)])

    @pl.when((cid == 1) & (sid < NS - 1))
    def _():
        pltpu.sync_copy(acc.at[pl.ds(node0, NODES_PER_TILE)],
                        o1.at[pl.ds(node0, NODES_PER_TILE)])

    @pl.when((cid == 1) & (sid == NS - 1))
    def _():
        pltpu.sync_copy(acc.at[pl.ds(node0, NODES_LAST_TILE)],
                        o1.at[pl.ds(node0, NODES_LAST_TILE)])


def kernel(x, edge_index, W):
    ei = edge_index.astype(jnp.int32)
    src = ei[1]
    dst = ei[0]
    ept = N_EDGES // NS
    padt = EDGES_PER_TILE - ept
    pad_src = jnp.broadcast_to(jnp.arange(padt, dtype=jnp.int32), (NS, padt))
    pad_dst = jnp.broadcast_to(
        N_NODES + (jnp.arange(padt, dtype=jnp.int32) % 16), (NS, padt))
    src3 = jnp.concatenate(
        [src.reshape(NS, ept), pad_src], axis=1).reshape(
            NS, CHUNKS_PER_TILE, CHUNK)
    dst3 = jnp.concatenate(
        [dst.reshape(NS, ept), pad_dst], axis=1).reshape(
            NS, CHUNKS_PER_TILE, CHUNK)
    wt = W.T
    h0, h1 = _linear(x, wt[:, :HALF], wt[:, HALF:])
    zeros = jnp.zeros((N_NODES, HALF), jnp.float32)
    o0, o1 = _aggregate(h0, h1, src3, dst3, zeros)
    return jnp.concatenate([o0, o1], axis=1)


# speedup vs baseline: 2.7539x; 1.0312x over previous
import functools

import jax
import jax.numpy as jnp
from jax import lax
from jax.experimental import pallas as pl
from jax.experimental.pallas import tpu as pltpu
from jax.experimental.pallas import tpu_sc as plsc

N_NODES = 10000
N_EDGES = 160000
IN_DIM = 512
OUT_DIM = 256
HALF = 128
NC = 2
NS = 16
CHUNK = 128
CHUNKS_PER_TILE = 80
PASSES = 2
CPP = CHUNKS_PER_TILE // PASSES
EDGES_PER_TILE = CHUNK * CHUNKS_PER_TILE
E_PAD = EDGES_PER_TILE * NS
NODES_PER_TILE = 624
NODES_LAST_TILE = N_NODES - NODES_PER_TILE * (NS - 1)
ACC_ROWS = N_NODES + 16
ROW_BLK = 2000


def _mm_body(x_ref, w0_ref, w1_ref, o0_ref, o1_ref):
    o0_ref[...] = jnp.dot(x_ref[...], w0_ref[...],
                          preferred_element_type=jnp.float32)
    o1_ref[...] = jnp.dot(x_ref[...], w1_ref[...],
                          preferred_element_type=jnp.float32)


def _linear(x, w0, w1):
    return pl.pallas_call(
        _mm_body,
        grid=(N_NODES // ROW_BLK,),
        in_specs=[
            pl.BlockSpec((ROW_BLK, IN_DIM), lambda r: (r, 0)),
            pl.BlockSpec((IN_DIM, HALF), lambda r: (0, 0)),
            pl.BlockSpec((IN_DIM, HALF), lambda r: (0, 0)),
        ],
        out_specs=[
            pl.BlockSpec((ROW_BLK, HALF), lambda r: (r, 0)),
            pl.BlockSpec((ROW_BLK, HALF), lambda r: (r, 0)),
        ],
        out_shape=[
            jax.ShapeDtypeStruct((N_NODES, HALF), jnp.float32),
            jax.ShapeDtypeStruct((N_NODES, HALF), jnp.float32),
        ],
    )(x, w0, w1)


_mesh = plsc.VectorSubcoreMesh(core_axis_name="c", subcore_axis_name="s",
                               num_cores=NC, num_subcores=NS)


@functools.partial(
    pl.kernel,
    out_type=(jax.ShapeDtypeStruct((N_NODES, HALF), jnp.float32),
              jax.ShapeDtypeStruct((N_NODES, HALF), jnp.float32)),
    mesh=_mesh,
    scratch_types=[
        pltpu.VMEM((CPP, CHUNK), jnp.int32),
        pltpu.VMEM((CPP, CHUNK), jnp.int32),
        pltpu.VMEM((CHUNK, HALF), jnp.float32),
        pltpu.VMEM((CHUNK, HALF), jnp.float32),
        pltpu.VMEM_SHARED((ACC_ROWS, HALF), jnp.float32),
        pltpu.SemaphoreType.DMA,
        pltpu.SemaphoreType.DMA,
    ],
)
def _aggregate(h0, h1, src3, dst3, o0, o1,
               src_v, dst_v, r0, r1, acc, g0, g1):
    bufs = (r0, r1)
    gsems = (g0, g1)
    cid = lax.axis_index("c")
    sid = lax.axis_index("s")
    node0 = sid * NODES_PER_TILE

    zvec = jnp.zeros((16,), jnp.float32)

    def zrow(i, carry):
        for k in range(8):
            r0[i, pl.ds(k * 16, 16)] = zvec
        return carry

    lax.fori_loop(0, CHUNK, zrow, 0)

    for q in range(4):
        pltpu.sync_copy(r0, acc.at[pl.ds(node0 + q * CHUNK, CHUNK)])

    @pl.when(sid < NS - 1)
    def _():
        pltpu.sync_copy(r0.at[pl.ds(0, 112)],
                        acc.at[pl.ds(node0 + 4 * CHUNK, 112)])

    @pl.when(sid == NS - 1)
    def _():
        pltpu.sync_copy(r0, acc.at[pl.ds(node0 + 4 * CHUNK, CHUNK)])

    plsc.subcore_barrier()

    def start_gather(jj, b):
        @pl.when(cid == 0)
        def _():
            pltpu.async_copy(h0.at[src_v.at[jj]], bufs[b], gsems[b])

        @pl.when(cid == 1)
        def _():
            pltpu.async_copy(h1.at[src_v.at[jj]], bufs[b], gsems[b])

    def wait_gather(b):
        pltpu.make_async_copy(h0.at[pl.ds(0, CHUNK)], bufs[b], gsems[b]).wait()

    for p in range(PASSES):
        pltpu.sync_copy(src3.at[sid, pl.ds(p * CPP, CPP)], src_v)
        pltpu.sync_copy(dst3.at[sid, pl.ds(p * CPP, CPP)], dst_v)
        start_gather(0, 0)

        def pair(i, carry):
            for b in range(2):
                j = 2 * i + b

                @pl.when(j + 1 < CPP)
                def _(j=j, b=b):
                    start_gather(j + 1, 1 - b)

                wait_gather(b)
                pltpu.sync_copy(bufs[b], acc.at[dst_v.at[j]], add=True)
            return carry

        lax.fori_loop(0, CPP // 2, pair, 0)

    plsc.subcore_barrier()

    @pl.when((cid == 0) & (sid < NS - 1))
    def _():
        pltpu.sync_copy(acc.at[pl.ds(node0, NODES_PER_TILE)],
                        o0.at[pl.ds(node0, NODES_PER_TILE)])

    @pl.when((cid == 0) & (sid == NS - 1))
    def _():
        pltpu.sync_copy(acc.at[pl.ds(node0, NODES_LAST_TILE)],
                        o0.at[pl.ds(node0, NODES_LAST_TILE)])

    @pl.when((cid == 1) & (sid < NS - 1))
    def _():
        pltpu.sync_copy(acc.at[pl.ds(node0, NODES_PER_TILE)],
                        o1.at[pl.ds(node0, NODES_PER_TILE)])

    @pl.when((cid == 1) & (sid == NS - 1))
    def _():
        pltpu.sync_copy(acc.at[pl.ds(node0, NODES_LAST_TILE)],
                        o1.at[pl.ds(node0, NODES_LAST_TILE)])


def kernel(x, edge_index, W):
    ei = edge_index.astype(jnp.int32)
    src = ei[1]
    dst = ei[0]
    ept = N_EDGES // NS
    padt = EDGES_PER_TILE - ept
    pad_src = jnp.broadcast_to(jnp.arange(padt, dtype=jnp.int32), (NS, padt))
    pad_dst = jnp.broadcast_to(
        N_NODES + (jnp.arange(padt, dtype=jnp.int32) % 16), (NS, padt))
    src3 = jnp.concatenate(
        [src.reshape(NS, ept), pad_src], axis=1).reshape(
            NS, CHUNKS_PER_TILE, CHUNK)
    dst3 = jnp.concatenate(
        [dst.reshape(NS, ept), pad_dst], axis=1).reshape(
            NS, CHUNKS_PER_TILE, CHUNK)
    wt = W.T
    h0, h1 = _linear(x, wt[:, :HALF], wt[:, HALF:])
    o0, o1 = _aggregate(h0, h1, src3, dst3)
    return jnp.concatenate([o0, o1], axis=1)


# speedup vs baseline: 2.7915x; 1.0137x over previous
import functools

import jax
import jax.numpy as jnp
from jax import lax
from jax.experimental import pallas as pl
from jax.experimental.pallas import tpu as pltpu
from jax.experimental.pallas import tpu_sc as plsc

N_NODES = 10000
N_EDGES = 160000
IN_DIM = 512
OUT_DIM = 256
HALF = 128
NC = 2
NS = 16
CHUNK = 128
CHUNKS_PER_TILE = 80
PASSES = 2
CPP = CHUNKS_PER_TILE // PASSES
EDGES_PER_TILE = CHUNK * CHUNKS_PER_TILE
E_PAD = EDGES_PER_TILE * NS
NODES_PER_TILE = 624
NODES_LAST_TILE = N_NODES - NODES_PER_TILE * (NS - 1)
ACC_ROWS = N_NODES + 16
ROW_BLK = 2000


def _mm_body(x_ref, w_ref, o0_ref, o1_ref):
    dn = (((1,), (1,)), ((), ()))
    o0_ref[...] = lax.dot_general(x_ref[...], w_ref[0:HALF, :], dn,
                                  preferred_element_type=jnp.float32)
    o1_ref[...] = lax.dot_general(x_ref[...], w_ref[HALF:OUT_DIM, :], dn,
                                  preferred_element_type=jnp.float32)


def _linear(x, w):
    return pl.pallas_call(
        _mm_body,
        grid=(N_NODES // ROW_BLK,),
        in_specs=[
            pl.BlockSpec((ROW_BLK, IN_DIM), lambda r: (r, 0)),
            pl.BlockSpec((OUT_DIM, IN_DIM), lambda r: (0, 0)),
        ],
        out_specs=[
            pl.BlockSpec((ROW_BLK, HALF), lambda r: (r, 0)),
            pl.BlockSpec((ROW_BLK, HALF), lambda r: (r, 0)),
        ],
        out_shape=[
            jax.ShapeDtypeStruct((N_NODES, HALF), jnp.float32),
            jax.ShapeDtypeStruct((N_NODES, HALF), jnp.float32),
        ],
    )(x, w)


_mesh = plsc.VectorSubcoreMesh(core_axis_name="c", subcore_axis_name="s",
                               num_cores=NC, num_subcores=NS)


@functools.partial(
    pl.kernel,
    out_type=(jax.ShapeDtypeStruct((N_NODES, HALF), jnp.float32),
              jax.ShapeDtypeStruct((N_NODES, HALF), jnp.float32)),
    mesh=_mesh,
    scratch_types=[
        pltpu.VMEM((CPP, CHUNK), jnp.int32),
        pltpu.VMEM((CPP, CHUNK), jnp.int32),
        pltpu.VMEM((CHUNK, HALF), jnp.float32),
        pltpu.VMEM((CHUNK, HALF), jnp.float32),
        pltpu.VMEM_SHARED((ACC_ROWS, HALF), jnp.float32),
        pltpu.SemaphoreType.DMA,
        pltpu.SemaphoreType.DMA,
    ],
)
def _aggregate(h0, h1, src3, dst3, o0, o1,
               src_v, dst_v, r0, r1, acc, g0, g1):
    bufs = (r0, r1)
    gsems = (g0, g1)
    cid = lax.axis_index("c")
    sid = lax.axis_index("s")
    node0 = sid * NODES_PER_TILE

    zvec = jnp.zeros((16,), jnp.float32)

    def zrow(i, carry):
        for k in range(8):
            r0[i, pl.ds(k * 16, 16)] = zvec
        return carry

    lax.fori_loop(0, CHUNK, zrow, 0)

    for q in range(4):
        pltpu.sync_copy(r0, acc.at[pl.ds(node0 + q * CHUNK, CHUNK)])

    @pl.when(sid < NS - 1)
    def _():
        pltpu.sync_copy(r0.at[pl.ds(0, 112)],
                        acc.at[pl.ds(node0 + 4 * CHUNK, 112)])

    @pl.when(sid == NS - 1)
    def _():
        pltpu.sync_copy(r0, acc.at[pl.ds(node0 + 4 * CHUNK, CHUNK)])

    plsc.subcore_barrier()

    def start_gather(jj, b):
        @pl.when(cid == 0)
        def _():
            pltpu.async_copy(h0.at[src_v.at[jj]], bufs[b], gsems[b])

        @pl.when(cid == 1)
        def _():
            pltpu.async_copy(h1.at[src_v.at[jj]], bufs[b], gsems[b])

    def wait_gather(b):
        pltpu.make_async_copy(h0.at[pl.ds(0, CHUNK)], bufs[b], gsems[b]).wait()

    for p in range(PASSES):
        pltpu.sync_copy(src3.at[sid, pl.ds(p * CPP, CPP)], src_v)
        pltpu.sync_copy(dst3.at[sid, pl.ds(p * CPP, CPP)], dst_v)
        start_gather(0, 0)

        def pair(i, carry):
            for b in range(2):
                j = 2 * i + b

                @pl.when(j + 1 < CPP)
                def _(j=j, b=b):
                    start_gather(j + 1, 1 - b)

                wait_gather(b)
                pltpu.sync_copy(bufs[b], acc.at[dst_v.at[j]], add=True)
            return carry

        lax.fori_loop(0, CPP // 2, pair, 0)

    plsc.subcore_barrier()

    @pl.when((cid == 0) & (sid < NS - 1))
    def _():
        pltpu.sync_copy(acc.at[pl.ds(node0, NODES_PER_TILE)],
                        o0.at[pl.ds(node0, NODES_PER_TILE)])

    @pl.when((cid == 0) & (sid == NS - 1))
    def _():
        pltpu.sync_copy(acc.at[pl.ds(node0, NODES_LAST_TILE)],
                        o0.at[pl.ds(node0, NODES_LAST_TILE)])

    @pl.when((cid == 1) & (sid < NS - 1))
    def _():
        pltpu.sync_copy(acc.at[pl.ds(node0, NODES_PER_TILE)],
                        o1.at[pl.ds(node0, NODES_PER_TILE)])

    @pl.when((cid == 1) & (sid == NS - 1))
    def _():
        pltpu.sync_copy(acc.at[pl.ds(node0, NODES_LAST_TILE)],
                        o1.at[pl.ds(node0, NODES_LAST_TILE)])


def kernel(x, edge_index, W):
    ei = edge_index.astype(jnp.int32)
    src = ei[1]
    dst = ei[0]
    ept = N_EDGES // NS
    padt = EDGES_PER_TILE - ept
    pad_src = jnp.broadcast_to(jnp.arange(padt, dtype=jnp.int32), (NS, padt))
    pad_dst = jnp.broadcast_to(
        N_NODES + (jnp.arange(padt, dtype=jnp.int32) % 16), (NS, padt))
    src3 = jnp.concatenate(
        [src.reshape(NS, ept), pad_src], axis=1).reshape(
            NS, CHUNKS_PER_TILE, CHUNK)
    dst3 = jnp.concatenate(
        [dst.reshape(NS, ept), pad_dst], axis=1).reshape(
            NS, CHUNKS_PER_TILE, CHUNK)
    h0, h1 = _linear(x, W)
    o0, o1 = _aggregate(h0, h1, src3, dst3)
    return jnp.concatenate([o0, o1], axis=1)


# speedup vs baseline: 2.8280x; 1.0131x over previous
import functools

import jax
import jax.numpy as jnp
from jax import lax
from jax.experimental import pallas as pl
from jax.experimental.pallas import tpu as pltpu
from jax.experimental.pallas import tpu_sc as plsc

N_NODES = 10000
N_EDGES = 160000
IN_DIM = 512
OUT_DIM = 256
HALF = 128
NC = 2
NS = 16
CHUNK = 128
CHUNKS_PER_TILE = 80
PASSES = 2
CPP = CHUNKS_PER_TILE // PASSES
EDGES_PER_TILE = CHUNK * CHUNKS_PER_TILE
E_PAD = EDGES_PER_TILE * NS
NODES_PER_TILE = 624
NODES_LAST_TILE = N_NODES - NODES_PER_TILE * (NS - 1)
ACC_ROWS = N_NODES + 16
ROW_BLK = 2000


def _mm_body(x_ref, w_ref, o0_ref, o1_ref):
    dn = (((1,), (1,)), ((), ()))
    o0_ref[...] = lax.dot_general(x_ref[...], w_ref[0:HALF, :], dn,
                                  preferred_element_type=jnp.float32)
    o1_ref[...] = lax.dot_general(x_ref[...], w_ref[HALF:OUT_DIM, :], dn,
                                  preferred_element_type=jnp.float32)


def _linear(x, w):
    return pl.pallas_call(
        _mm_body,
        grid=(N_NODES // ROW_BLK,),
        in_specs=[
            pl.BlockSpec((ROW_BLK, IN_DIM), lambda r: (r, 0)),
            pl.BlockSpec((OUT_DIM, IN_DIM), lambda r: (0, 0)),
        ],
        out_specs=[
            pl.BlockSpec((ROW_BLK, HALF), lambda r: (r, 0)),
            pl.BlockSpec((ROW_BLK, HALF), lambda r: (r, 0)),
        ],
        out_shape=[
            jax.ShapeDtypeStruct((N_NODES, HALF), jnp.float32),
            jax.ShapeDtypeStruct((N_NODES, HALF), jnp.float32),
        ],
    )(x, w)


_mesh = plsc.VectorSubcoreMesh(core_axis_name="c", subcore_axis_name="s",
                               num_cores=NC, num_subcores=NS)


@functools.partial(
    pl.kernel,
    out_type=(jax.ShapeDtypeStruct((N_NODES, HALF), jnp.float32),
              jax.ShapeDtypeStruct((N_NODES, HALF), jnp.float32)),
    mesh=_mesh,
    scratch_types=[
        pltpu.VMEM((CPP, CHUNK), jnp.int32),
        pltpu.VMEM((CPP, CHUNK), jnp.int32),
        pltpu.VMEM((CHUNK, HALF), jnp.float32),
        pltpu.VMEM((CHUNK, HALF), jnp.float32),
        pltpu.VMEM_SHARED((ACC_ROWS, HALF), jnp.float32),
        pltpu.SemaphoreType.DMA,
        pltpu.SemaphoreType.DMA,
    ],
)
def _aggregate(h0, h1, src3, dst3, o0, o1,
               src_v, dst_v, r0, r1, acc, g0, g1):
    bufs = (r0, r1)
    gsems = (g0, g1)
    cid = lax.axis_index("c")
    sid = lax.axis_index("s")
    node0 = sid * NODES_PER_TILE

    def start_gather(jj, b):
        @pl.when(cid == 0)
        def _():
            pltpu.async_copy(h0.at[src_v.at[jj]], bufs[b], gsems[b])

        @pl.when(cid == 1)
        def _():
            pltpu.async_copy(h1.at[src_v.at[jj]], bufs[b], gsems[b])

    def wait_gather(b):
        pltpu.make_async_copy(h0.at[pl.ds(0, CHUNK)], bufs[b], gsems[b]).wait()

    pltpu.sync_copy(src3.at[sid, pl.ds(0, CPP)], src_v)
    pltpu.sync_copy(dst3.at[sid, pl.ds(0, CPP)], dst_v)
    start_gather(0, 0)

    zvec = jnp.zeros((16,), jnp.float32)

    def zrow(i, carry):
        for k in range(8):
            r1[i, pl.ds(k * 16, 16)] = zvec
        return carry

    lax.fori_loop(0, CHUNK, zrow, 0)

    for q in range(4):
        pltpu.sync_copy(r1, acc.at[pl.ds(node0 + q * CHUNK, CHUNK)])

    @pl.when(sid < NS - 1)
    def _():
        pltpu.sync_copy(r1.at[pl.ds(0, 112)],
                        acc.at[pl.ds(node0 + 4 * CHUNK, 112)])

    @pl.when(sid == NS - 1)
    def _():
        pltpu.sync_copy(r1, acc.at[pl.ds(node0 + 4 * CHUNK, CHUNK)])

    plsc.subcore_barrier()

    for p in range(PASSES):
        if p > 0:
            pltpu.sync_copy(src3.at[sid, pl.ds(p * CPP, CPP)], src_v)
            pltpu.sync_copy(dst3.at[sid, pl.ds(p * CPP, CPP)], dst_v)
            start_gather(0, 0)

        def pair(i, carry):
            for b in range(2):
                j = 2 * i + b

                @pl.when(j + 1 < CPP)
                def _(j=j, b=b):
                    start_gather(j + 1, 1 - b)

                wait_gather(b)
                pltpu.sync_copy(bufs[b], acc.at[dst_v.at[j]], add=True)
            return carry

        lax.fori_loop(0, CPP // 2, pair, 0)

    plsc.subcore_barrier()

    @pl.when((cid == 0) & (sid < NS - 1))
    def _():
        pltpu.sync_copy(acc.at[pl.ds(node0, NODES_PER_TILE)],
                        o0.at[pl.ds(node0, NODES_PER_TILE)])

    @pl.when((cid == 0) & (sid == NS - 1))
    def _():
        pltpu.sync_copy(acc.at[pl.ds(node0, NODES_LAST_TILE)],
                        o0.at[pl.ds(node0, NODES_LAST_TILE)])

    @pl.when((cid == 1) & (sid < NS - 1))
    def _():
        pltpu.sync_copy(acc.at[pl.ds(node0, NODES_PER_TILE)],
                        o1.at[pl.ds(node0, NODES_PER_TILE)])

    @pl.when((cid == 1) & (sid == NS - 1))
    def _():
        pltpu.sync_copy(acc.at[pl.ds(node0, NODES_LAST_TILE)],
                        o1.at[pl.ds(node0, NODES_LAST_TILE)])


def kernel(x, edge_index, W):
    ei = edge_index.astype(jnp.int32)
    src = ei[1]
    dst = ei[0]
    ept = N_EDGES // NS
    padt = EDGES_PER_TILE - ept
    pad_src = jnp.broadcast_to(jnp.arange(padt, dtype=jnp.int32), (NS, padt))
    pad_dst = jnp.broadcast_to(
        N_NODES + (jnp.arange(padt, dtype=jnp.int32) % 16), (NS, padt))
    src3 = jnp.concatenate(
        [src.reshape(NS, ept), pad_src], axis=1).reshape(
            NS, CHUNKS_PER_TILE, CHUNK)
    dst3 = jnp.concatenate(
        [dst.reshape(NS, ept), pad_dst], axis=1).reshape(
            NS, CHUNKS_PER_TILE, CHUNK)
    h0, h1 = _linear(x, W)
    o0, o1 = _aggregate(h0, h1, src3, dst3)
    return jnp.concatenate([o0, o1], axis=1)
